# baseline (speedup 1.0000x reference)
import functools

import jax
import jax.numpy as jnp
from jax import lax
from jax.experimental import pallas as pl
from jax.experimental.pallas import tpu as pltpu
from jax.experimental.pallas import tpu_sc as plsc

B, S = 4, 8192
V, D = 256, 1024
N = B * S

NC, NS = 2, 16
NW = NC * NS
NQ = 4
DQ = D // NQ
NG = NW // NQ
ROWS_PER_G = N // NG
C = 64
NCHUNK = ROWS_PER_G // C

_mesh = plsc.VectorSubcoreMesh(core_axis_name="c", subcore_axis_name="s")


@functools.partial(
    pl.kernel,
    mesh=_mesh,
    out_type=jax.ShapeDtypeStruct((N, D), jnp.float32),
    scratch_types=[
        pltpu.VMEM((NCHUNK, C), jnp.int32),
        pltpu.VMEM((V, DQ), jnp.float32),
        pltpu.VMEM((C, DQ), jnp.float32),
        pltpu.VMEM((C, DQ), jnp.float32),
        pltpu.SemaphoreType.DMA,
        pltpu.SemaphoreType.DMA,
    ],
)
def _sc_gather(idx_hbm, shards_hbm, out_hbm, idx_v, table_v, buf0, buf1,
               wsem0, wsem1):
    wid = lax.axis_index("s") * NC + lax.axis_index("c")
    q = wid % NQ
    g = wid // NQ

    pltpu.sync_copy(shards_hbm.at[q], table_v)
    pltpu.sync_copy(idx_hbm.at[g], idx_v)

    bufs = (buf0, buf1)
    wsems = (wsem0, wsem1)
    col0 = q * DQ

    def copy_chunk(c, buf):
        def one_group(gi, carry):
            vec = idx_v[c, pl.ds(gi * 16, 16)]
            for l in range(16):
                r = vec[l]
                p = gi * 16 + l
                for j in range(DQ // 16):
                    buf[p, pl.ds(j * 16, 16)] = table_v[r, pl.ds(j * 16, 16)]
            return carry
        lax.fori_loop(0, C // 16, one_group, 0)

    def wait_write(buf, wsem):
        pltpu.make_async_copy(
            buf, out_hbm.at[pl.ds(0, C), pl.ds(col0, DQ)], wsem).wait()

    def outer(i2, carry):
        for b in range(2):
            c = i2 * 2 + b

            @pl.when(c >= 2)
            def _():
                wait_write(bufs[b], wsems[b])

            copy_chunk(c, bufs[b])
            row0 = g * ROWS_PER_G + c * C
            pltpu.async_copy(
                bufs[b], out_hbm.at[pl.ds(row0, C), pl.ds(col0, DQ)],
                wsems[b])
        return carry

    lax.fori_loop(0, NCHUNK // 2, outer, 0)
    wait_write(buf0, wsem0)
    wait_write(buf1, wsem1)


def kernel(input_ids, attention_mask, embed):
    idx = input_ids.reshape(NG, NCHUNK, C).astype(jnp.int32)
    shards = embed.reshape(V, NQ, DQ).transpose(1, 0, 2)
    out = _sc_gather(idx, shards)
    return out.reshape(B, S, D)
